# Initial kernel scaffold; baseline (speedup 1.0000x reference)
import functools

import jax
import jax.numpy as jnp
from jax import lax
from jax.experimental import pallas as pl
from jax.experimental.pallas import tpu as pltpu
from jax.experimental.pallas import tpu_sc as plsc

N = 10000
E = 320000
D = 128
BS = 1024
L = 50
ITEM_NUM = 9000
POS = 200

NC = 2
NS = 16
NW = NC * NS
EPW = E // NW
CH = 80
NCH = EPW // CH
RPT = N // NS

SEQ_T = BS * L
SPW = SEQ_T // NW
SCH = SPW // CH
UPW = BS // NW


def _edge_agg_body(src_hbm, dst_hbm, v2e_hbm, zrow_hbm, ones_hbm,
                   agg_out, deg_out,
                   src_v, dst_v, rows_v, ones_v, sem,
                   agg_sh, deg_sh):
    c = lax.axis_index("c")
    s = lax.axis_index("s")
    wid = s * NC + c

    base_r = s * RPT
    pltpu.sync_copy(zrow_hbm, agg_sh.at[pl.ds(base_r, RPT)])
    pltpu.sync_copy(zrow_hbm.at[:, pl.ds(0, 16)], deg_sh.at[pl.ds(base_r, RPT)])
    pltpu.sync_copy(ones_hbm, ones_v)
    plsc.subcore_barrier()

    def body(i, carry):
        off = wid * EPW + i * CH
        pltpu.sync_copy(src_hbm.at[pl.ds(off, CH)], src_v)
        pltpu.sync_copy(dst_hbm.at[pl.ds(off, CH)], dst_v)
        pltpu.async_copy(v2e_hbm.at[src_v], rows_v, sem).wait()
        pltpu.sync_copy(rows_v, agg_sh.at[dst_v], add=True)
        pltpu.sync_copy(ones_v, deg_sh.at[dst_v], add=True)
        return carry

    lax.fori_loop(0, NCH, body, 0)
    plsc.subcore_barrier()

    pltpu.sync_copy(agg_sh.at[pl.ds(base_r, RPT)],
                    agg_out.at[c, pl.ds(base_r, RPT)])
    pltpu.sync_copy(deg_sh.at[pl.ds(base_r, RPT)],
                    deg_out.at[c, pl.ds(base_r, RPT)])


def _edge_agg(src, dst, v2e, zrow, ones16):
    return pl.kernel(
        _edge_agg_body,
        out_type=[
            jax.ShapeDtypeStruct((NC, N, D), jnp.float32),
            jax.ShapeDtypeStruct((NC, N, 16), jnp.float32),
        ],
        mesh=plsc.VectorSubcoreMesh(core_axis_name="c", subcore_axis_name="s",
                                    num_cores=NC, num_subcores=NS),
        scratch_types=[
            pltpu.VMEM((CH,), jnp.int32),
            pltpu.VMEM((CH,), jnp.int32),
            pltpu.VMEM((CH, D), jnp.float32),
            pltpu.VMEM((CH, 16), jnp.float32),
            pltpu.SemaphoreType.DMA,
            pltpu.VMEM_SHARED((N, D), jnp.float32),
            pltpu.VMEM_SHARED((N, 16), jnp.float32),
        ],
    )(src, dst, v2e, zrow, ones16)


def _gather_body(g_hbm, pos_tab_hbm, seq_hbm, pos_hbm, uidx_hbm,
                 seqg_out, posg_out, userg_out,
                 idx_v, rows_v, idx2_v, rows2_v, uidx_v, urows_v, sem):
    c = lax.axis_index("c")
    s = lax.axis_index("s")
    wid = s * NC + c

    def body(i, carry):
        off = wid * SPW + i * CH
        pltpu.sync_copy(seq_hbm.at[pl.ds(off, CH)], idx_v)
        pltpu.async_copy(g_hbm.at[idx_v], rows_v, sem).wait()
        pltpu.sync_copy(rows_v, seqg_out.at[pl.ds(off, CH)])
        pltpu.sync_copy(pos_hbm.at[pl.ds(off, CH)], idx2_v)
        pltpu.async_copy(pos_tab_hbm.at[idx2_v], rows2_v, sem).wait()
        pltpu.sync_copy(rows2_v, posg_out.at[pl.ds(off, CH)])
        return carry

    lax.fori_loop(0, SCH, body, 0)

    uoff = wid * UPW
    pltpu.sync_copy(uidx_hbm.at[pl.ds(uoff, UPW)], uidx_v)
    pltpu.async_copy(g_hbm.at[uidx_v], urows_v, sem).wait()
    pltpu.sync_copy(urows_v, userg_out.at[pl.ds(uoff, UPW)])


def _gathers(g, pos_table, seq_flat, pos_flat, uidx):
    return pl.kernel(
        _gather_body,
        out_type=[
            jax.ShapeDtypeStruct((SEQ_T, D), jnp.float32),
            jax.ShapeDtypeStruct((SEQ_T, D), jnp.float32),
            jax.ShapeDtypeStruct((BS, D), jnp.float32),
        ],
        mesh=plsc.VectorSubcoreMesh(core_axis_name="c", subcore_axis_name="s",
                                    num_cores=NC, num_subcores=NS),
        scratch_types=[
            pltpu.VMEM((CH,), jnp.int32),
            pltpu.VMEM((CH, D), jnp.float32),
            pltpu.VMEM((CH,), jnp.int32),
            pltpu.VMEM((CH, D), jnp.float32),
            pltpu.VMEM((UPW,), jnp.int32),
            pltpu.VMEM((UPW, D), jnp.float32),
            pltpu.SemaphoreType.DMA,
        ],
    )(g, pos_table, seq_flat, pos_flat, uidx)


_BN = 400


def _sage_body(v2e_ref, agg_ref, deg_ref, ws_ref, wn_ref, b_ref, g_ref):
    x = v2e_ref[...]
    a = agg_ref[0] + agg_ref[1]
    d = deg_ref[0][:, 0:1] + deg_ref[1][:, 0:1]
    neigh = a / jnp.maximum(d, 1.0)
    h = jnp.dot(x, ws_ref[...], preferred_element_type=jnp.float32)
    h = h + jnp.dot(neigh, wn_ref[...], preferred_element_type=jnp.float32)
    h = jax.nn.relu(h + b_ref[...])
    g_ref[...] = 0.5 * (h + x)


def _sage(v2e, agg, deg, Wself, Wneigh, bneigh):
    grid = (N // _BN,)
    return pl.pallas_call(
        _sage_body,
        grid=grid,
        in_specs=[
            pl.BlockSpec((_BN, D), lambda i: (i, 0)),
            pl.BlockSpec((NC, _BN, D), lambda i: (0, i, 0)),
            pl.BlockSpec((NC, _BN, 16), lambda i: (0, i, 0)),
            pl.BlockSpec((D, D), lambda i: (0, 0)),
            pl.BlockSpec((D, D), lambda i: (0, 0)),
            pl.BlockSpec((1, D), lambda i: (0, 0)),
        ],
        out_specs=pl.BlockSpec((_BN, D), lambda i: (i, 0)),
        out_shape=jax.ShapeDtypeStruct((N, D), jnp.float32),
    )(v2e, agg, deg, Wself, Wneigh, bneigh)


_BB = 8
_BM = _BB * L


def _attn_body(node_ref, pos_ref, m_ref, u_ref,
               w1a_ref, w1b_ref, g1w_ref, g1b_ref, g2w_ref, w2_ref,
               w3_ref, g3w_ref, g3b_ref, g4w_ref, w4_ref,
               scwa_ref, scwb_ref, scb_ref, out_ref):
    node = node_ref[...]
    posv = pos_ref[...]
    m = m_ref[...]
    u = u_ref[...]

    row = lax.broadcasted_iota(jnp.int32, (_BM, _BB), 0) // L
    col = lax.broadcasted_iota(jnp.int32, (_BM, _BB), 1)
    selT = (row == col).astype(jnp.float32)

    def seg_sum(x):
        return lax.dot_general(selT, x, (((0,), (0,)), ((), ())),
                               preferred_element_type=jnp.float32)

    def expand(x):
        return jnp.dot(selT, x, preferred_element_type=jnp.float32)

    mnode = node * m
    tmp = seg_sum(mnode) / seg_sum(m)
    hsb = expand(jnp.dot(tmp, g2w_ref[...],
                         preferred_element_type=jnp.float32))

    nh = jnp.tanh(jnp.dot(posv, w1a_ref[...], preferred_element_type=jnp.float32)
                  + jnp.dot(node, w1b_ref[...], preferred_element_type=jnp.float32))
    nh = jax.nn.sigmoid(jnp.dot(nh, g1w_ref[...], preferred_element_type=jnp.float32)
                        + g1b_ref[...] + hsb)
    beta = jnp.sum(nh * w2_ref[...], axis=-1, keepdims=True) * m
    sess = seg_sum(beta * node)

    ub = expand(jnp.dot(u, g4w_ref[...], preferred_element_type=jnp.float32))
    nh2 = jnp.tanh(jnp.dot(node, w3_ref[...], preferred_element_type=jnp.float32))
    nh2 = jax.nn.sigmoid(jnp.dot(nh2, g3w_ref[...], preferred_element_type=jnp.float32)
                         + g3b_ref[...] + ub)
    beta2 = jnp.sum(nh2 * w4_ref[...], axis=-1, keepdims=True) * m
    sessu = seg_sum(beta2 * node)

    a1 = jnp.sum(sess * scwa_ref[...], axis=-1, keepdims=True)
    a2 = jnp.sum(sessu * scwb_ref[...], axis=-1, keepdims=True)
    alpha = jax.nn.sigmoid(a1 + a2 + scb_ref[0:1, 0:1])
    out_ref[...] = u + alpha * sess + (1.0 - alpha) * sessu


def _attn(seqg, posg, maskf, userg, w1a, w1b, g1w, g1b, g2w, w2r,
          w3, g3w, g3b, g4w, w4r, scwa, scwb, scb):
    full = lambda shape: pl.BlockSpec(shape, lambda i: tuple(0 for _ in shape))
    return pl.pallas_call(
        _attn_body,
        grid=(BS // _BB,),
        in_specs=[
            pl.BlockSpec((_BM, D), lambda i: (i, 0)),
            pl.BlockSpec((_BM, D), lambda i: (i, 0)),
            pl.BlockSpec((_BM, 1), lambda i: (i, 0)),
            pl.BlockSpec((_BB, D), lambda i: (i, 0)),
            full((D, D)), full((D, D)), full((D, D)), full((1, D)),
            full((D, D)), full((1, D)),
            full((D, D)), full((D, D)), full((1, D)), full((D, D)),
            full((1, D)), full((1, D)), full((1, D)), full((1, D)),
        ],
        out_specs=pl.BlockSpec((_BB, D), lambda i: (i, 0)),
        out_shape=jax.ShapeDtypeStruct((BS, D), jnp.float32),
    )(seqg, posg, maskf, userg, w1a, w1b, g1w, g1b, g2w, w2r,
      w3, g3w, g3b, g4w, w4r, scwa, scwb, scb)


_NP = 10240
_BV = 1280


def _scores_body(f_ref, v_ref, o_ref):
    o_ref[...] = lax.dot_general(f_ref[...], v_ref[...],
                                 (((1,), (1,)), ((), ())),
                                 preferred_element_type=jnp.float32)


def _scores(femb, v2e_pad):
    return pl.pallas_call(
        _scores_body,
        grid=(_NP // _BV,),
        in_specs=[
            pl.BlockSpec((BS, D), lambda i: (0, 0)),
            pl.BlockSpec((_BV, D), lambda i: (i, 0)),
        ],
        out_specs=pl.BlockSpec((BS, _BV), lambda i: (0, i)),
        out_shape=jax.ShapeDtypeStruct((BS, _NP), jnp.float32),
    )(femb, v2e_pad)


def kernel(user, seq, mask, seq_len, pos_idx, edge_index, v2e, pos_table,
           Wself, Wneigh, bneigh, w1, w2, glu1_W, glu1_b, glu2_W, w3, w4,
           glu3_W, glu3_b, glu4_W, sc_W, sc_b):
    src = edge_index[0].astype(jnp.int32)
    dst = edge_index[1].astype(jnp.int32)
    zrow = jnp.zeros((RPT, D), jnp.float32)
    ones16 = jnp.ones((CH, 16), jnp.float32)

    agg, deg = _edge_agg(src, dst, v2e, zrow, ones16)
    g = _sage(v2e, agg, deg, Wself, Wneigh, bneigh.reshape(1, D))

    seq_flat = seq.reshape(SEQ_T).astype(jnp.int32)
    pos_flat = pos_idx.reshape(SEQ_T).astype(jnp.int32)
    uidx = (user[:, 0] + ITEM_NUM).astype(jnp.int32)
    seqg, posg, userg = _gathers(g, pos_table, seq_flat, pos_flat, uidx)

    maskf = mask.astype(jnp.float32).reshape(SEQ_T, 1)
    femb = _attn(
        seqg, posg, maskf, userg,
        w1[:D], w1[D:], glu1_W, glu1_b.reshape(1, D), glu2_W,
        w2.reshape(1, D), w3, glu3_W, glu3_b.reshape(1, D), glu4_W,
        w4.reshape(1, D), sc_W[:D].reshape(1, D), sc_W[D:].reshape(1, D),
        jnp.broadcast_to(sc_b.reshape(1, 1), (1, D)),
    )

    v2e_pad = jnp.pad(v2e, ((0, _NP - N), (0, 0)))
    scores = _scores(femb, v2e_pad)
    return scores[:, 1:N]


# speedup vs baseline: 2.7125x; 2.7125x over previous
import functools

import jax
import jax.numpy as jnp
from jax import lax
from jax.experimental import pallas as pl
from jax.experimental.pallas import tpu as pltpu
from jax.experimental.pallas import tpu_sc as plsc

N = 10000
E = 320000
D = 128
BS = 1024
L = 50
ITEM_NUM = 9000
POS = 200

NC = 2
NS = 16
NW = NC * NS
EPW = E // NW
CH = 80
NCH = EPW // CH
NPAD = 10240
RPT = NPAD // NS
RCH = RPT // CH

SEQ_T = BS * L
SPW = SEQ_T // NW
SCH = SPW // CH
UPW = BS // NW


def _edge_agg_body(src_hbm, dst_hbm, v2e_hbm, z80_hbm,
                   agg_out, deg_out,
                   src_v, dst_v, rows_v, deg_v, sem,
                   agg_sh):
    c = lax.axis_index("c")
    s = lax.axis_index("s")
    wid = s * NC + c
    base_r = s * RPT

    pltpu.sync_copy(z80_hbm, rows_v)

    def zbody(j, carry):
        pltpu.sync_copy(rows_v, agg_sh.at[pl.ds(base_r + j * CH, CH)])
        return carry
    lax.fori_loop(0, RCH, zbody, 0)

    zero16 = jnp.zeros((16,), jnp.float32)
    ones16 = jnp.ones((16,), jnp.float32)

    def zdeg(j, carry):
        deg_v[pl.ds(j * 16, 16)] = zero16
        return carry
    lax.fori_loop(0, NPAD // 16, zdeg, 0)
    plsc.subcore_barrier()

    def ebody(i, carry):
        off = wid * EPW + i * CH
        pltpu.sync_copy(src_hbm.at[pl.ds(off, CH)], src_v)
        pltpu.sync_copy(dst_hbm.at[pl.ds(off, CH)], dst_v)
        pltpu.async_copy(v2e_hbm.at[src_v], rows_v, sem).wait()
        pltpu.sync_copy(rows_v, agg_sh.at[dst_v], add=True)
        for k in range(CH // 16):
            idx16 = dst_v[pl.ds(k * 16, 16)]
            plsc.addupdate_scatter(deg_v, [idx16], ones16)
        return carry
    lax.fori_loop(0, NCH, ebody, 0)
    plsc.subcore_barrier()

    out_r = c * NPAD + base_r

    def wbody(j, carry):
        pltpu.sync_copy(agg_sh.at[pl.ds(base_r + j * CH, CH)], rows_v)
        pltpu.sync_copy(rows_v, agg_out.at[pl.ds(out_r + j * CH, CH)])
        return carry
    lax.fori_loop(0, RCH, wbody, 0)
    pltpu.sync_copy(deg_v, deg_out.at[pl.ds(wid * NPAD, NPAD)])


def _edge_agg(src, dst, v2e, z80):
    return pl.kernel(
        _edge_agg_body,
        out_type=[
            jax.ShapeDtypeStruct((NC * NPAD, D), jnp.float32),
            jax.ShapeDtypeStruct((NW * NPAD,), jnp.float32),
        ],
        mesh=plsc.VectorSubcoreMesh(core_axis_name="c", subcore_axis_name="s",
                                    num_cores=NC, num_subcores=NS),
        scratch_types=[
            pltpu.VMEM((CH,), jnp.int32),
            pltpu.VMEM((CH,), jnp.int32),
            pltpu.VMEM((CH, D), jnp.float32),
            pltpu.VMEM((NPAD,), jnp.float32),
            pltpu.SemaphoreType.DMA,
            pltpu.VMEM_SHARED((NPAD, D), jnp.float32),
        ],
        compiler_params=pltpu.CompilerParams(needs_layout_passes=False),
    )(src, dst, v2e, z80)


def _gather_body(g_hbm, pos_tab_hbm, seq_hbm, pos_hbm, uidx_hbm,
                 seqg_out, posg_out, userg_out,
                 idx_v, rows_v, idx2_v, rows2_v, uidx_v, urows_v, sem):
    c = lax.axis_index("c")
    s = lax.axis_index("s")
    wid = s * NC + c

    def body(i, carry):
        off = wid * SPW + i * CH
        pltpu.sync_copy(seq_hbm.at[pl.ds(off, CH)], idx_v)
        pltpu.async_copy(g_hbm.at[idx_v], rows_v, sem).wait()
        pltpu.sync_copy(rows_v, seqg_out.at[pl.ds(off, CH)])
        pltpu.sync_copy(pos_hbm.at[pl.ds(off, CH)], idx2_v)
        pltpu.async_copy(pos_tab_hbm.at[idx2_v], rows2_v, sem).wait()
        pltpu.sync_copy(rows2_v, posg_out.at[pl.ds(off, CH)])
        return carry

    lax.fori_loop(0, SCH, body, 0)

    uoff = wid * UPW
    pltpu.sync_copy(uidx_hbm.at[pl.ds(uoff, UPW)], uidx_v)
    pltpu.async_copy(g_hbm.at[uidx_v], urows_v, sem).wait()
    pltpu.sync_copy(urows_v, userg_out.at[pl.ds(uoff, UPW)])


def _gathers(g, pos_table, seq_flat, pos_flat, uidx):
    return pl.kernel(
        _gather_body,
        out_type=[
            jax.ShapeDtypeStruct((SEQ_T, D), jnp.float32),
            jax.ShapeDtypeStruct((SEQ_T, D), jnp.float32),
            jax.ShapeDtypeStruct((BS, D), jnp.float32),
        ],
        mesh=plsc.VectorSubcoreMesh(core_axis_name="c", subcore_axis_name="s",
                                    num_cores=NC, num_subcores=NS),
        scratch_types=[
            pltpu.VMEM((CH,), jnp.int32),
            pltpu.VMEM((CH, D), jnp.float32),
            pltpu.VMEM((CH,), jnp.int32),
            pltpu.VMEM((CH, D), jnp.float32),
            pltpu.VMEM((UPW,), jnp.int32),
            pltpu.VMEM((UPW, D), jnp.float32),
            pltpu.SemaphoreType.DMA,
        ],
    )(g, pos_table, seq_flat, pos_flat, uidx)


_BN = 400


def _sage_body(v2e_ref, agg_ref, deg_ref, ws_ref, wn_ref, b_ref, g_ref):
    x = v2e_ref[...]
    a = agg_ref[0] + agg_ref[1]
    d = jnp.sum(deg_ref[...], axis=0)
    neigh = a / jnp.maximum(d, 1.0)
    h = jnp.dot(x, ws_ref[...], preferred_element_type=jnp.float32)
    h = h + jnp.dot(neigh, wn_ref[...], preferred_element_type=jnp.float32)
    h = jax.nn.relu(h + b_ref[...])
    g_ref[...] = 0.5 * (h + x)


def _sage(v2e, agg, deg, Wself, Wneigh, bneigh):
    return pl.pallas_call(
        _sage_body,
        grid=(N // _BN,),
        in_specs=[
            pl.BlockSpec((_BN, D), lambda i: (i, 0)),
            pl.BlockSpec((NC, _BN, D), lambda i: (0, i, 0)),
            pl.BlockSpec((NW, _BN, 1), lambda i: (0, i, 0)),
            pl.BlockSpec((D, D), lambda i: (0, 0)),
            pl.BlockSpec((D, D), lambda i: (0, 0)),
            pl.BlockSpec((1, D), lambda i: (0, 0)),
        ],
        out_specs=pl.BlockSpec((_BN, D), lambda i: (i, 0)),
        out_shape=jax.ShapeDtypeStruct((N, D), jnp.float32),
    )(v2e, agg, deg, Wself, Wneigh, bneigh)


_BB = 8
_BM = _BB * L


def _attn_body(node_ref, pos_ref, m_ref, u_ref,
               w1a_ref, w1b_ref, g1w_ref, g1b_ref, g2w_ref, w2_ref,
               w3_ref, g3w_ref, g3b_ref, g4w_ref, w4_ref,
               scwa_ref, scwb_ref, scb_ref, out_ref):
    node = node_ref[...]
    posv = pos_ref[...]
    m = m_ref[...]
    u = u_ref[...]

    row = lax.broadcasted_iota(jnp.int32, (_BM, _BB), 0) // L
    col = lax.broadcasted_iota(jnp.int32, (_BM, _BB), 1)
    selT = (row == col).astype(jnp.float32)

    def seg_sum(x):
        return lax.dot_general(selT, x, (((0,), (0,)), ((), ())),
                               preferred_element_type=jnp.float32)

    def expand(x):
        return jnp.dot(selT, x, preferred_element_type=jnp.float32)

    mnode = node * m
    tmp = seg_sum(mnode) / seg_sum(m)
    hsb = expand(jnp.dot(tmp, g2w_ref[...],
                         preferred_element_type=jnp.float32))

    nh = jnp.tanh(jnp.dot(posv, w1a_ref[...], preferred_element_type=jnp.float32)
                  + jnp.dot(node, w1b_ref[...], preferred_element_type=jnp.float32))
    nh = jax.nn.sigmoid(jnp.dot(nh, g1w_ref[...], preferred_element_type=jnp.float32)
                        + g1b_ref[...] + hsb)
    beta = jnp.sum(nh * w2_ref[...], axis=-1, keepdims=True) * m
    sess = seg_sum(beta * node)

    ub = expand(jnp.dot(u, g4w_ref[...], preferred_element_type=jnp.float32))
    nh2 = jnp.tanh(jnp.dot(node, w3_ref[...], preferred_element_type=jnp.float32))
    nh2 = jax.nn.sigmoid(jnp.dot(nh2, g3w_ref[...], preferred_element_type=jnp.float32)
                         + g3b_ref[...] + ub)
    beta2 = jnp.sum(nh2 * w4_ref[...], axis=-1, keepdims=True) * m
    sessu = seg_sum(beta2 * node)

    a1 = jnp.sum(sess * scwa_ref[...], axis=-1, keepdims=True)
    a2 = jnp.sum(sessu * scwb_ref[...], axis=-1, keepdims=True)
    alpha = jax.nn.sigmoid(a1 + a2 + scb_ref[0:1, 0:1])
    out_ref[...] = u + alpha * sess + (1.0 - alpha) * sessu


def _attn(seqg, posg, maskf, userg, w1a, w1b, g1w, g1b, g2w, w2r,
          w3, g3w, g3b, g4w, w4r, scwa, scwb, scb):
    full = lambda shape: pl.BlockSpec(shape, lambda i: tuple(0 for _ in shape))
    return pl.pallas_call(
        _attn_body,
        grid=(BS // _BB,),
        in_specs=[
            pl.BlockSpec((_BM, D), lambda i: (i, 0)),
            pl.BlockSpec((_BM, D), lambda i: (i, 0)),
            pl.BlockSpec((_BM, 1), lambda i: (i, 0)),
            pl.BlockSpec((_BB, D), lambda i: (i, 0)),
            full((D, D)), full((D, D)), full((D, D)), full((1, D)),
            full((D, D)), full((1, D)),
            full((D, D)), full((D, D)), full((1, D)), full((D, D)),
            full((1, D)), full((1, D)), full((1, D)), full((1, D)),
        ],
        out_specs=pl.BlockSpec((_BB, D), lambda i: (i, 0)),
        out_shape=jax.ShapeDtypeStruct((BS, D), jnp.float32),
    )(seqg, posg, maskf, userg, w1a, w1b, g1w, g1b, g2w, w2r,
      w3, g3w, g3b, g4w, w4r, scwa, scwb, scb)


_NP = 10240
_BV = 1280


def _scores_body(f_ref, v_ref, o_ref):
    o_ref[...] = lax.dot_general(f_ref[...], v_ref[...],
                                 (((1,), (1,)), ((), ())),
                                 preferred_element_type=jnp.float32)


def _scores(femb, v2e_pad):
    return pl.pallas_call(
        _scores_body,
        grid=(_NP // _BV,),
        in_specs=[
            pl.BlockSpec((BS, D), lambda i: (0, 0)),
            pl.BlockSpec((_BV, D), lambda i: (i, 0)),
        ],
        out_specs=pl.BlockSpec((BS, _BV), lambda i: (0, i)),
        out_shape=jax.ShapeDtypeStruct((BS, _NP), jnp.float32),
    )(femb, v2e_pad)


def kernel(user, seq, mask, seq_len, pos_idx, edge_index, v2e, pos_table,
           Wself, Wneigh, bneigh, w1, w2, glu1_W, glu1_b, glu2_W, w3, w4,
           glu3_W, glu3_b, glu4_W, sc_W, sc_b):
    src = edge_index[0].astype(jnp.int32)
    dst = edge_index[1].astype(jnp.int32)
    z80 = jnp.zeros((CH, D), jnp.float32)

    agg, deg = _edge_agg(src, dst, v2e, z80)
    agg = agg.reshape(NC, NPAD, D)
    deg = deg.reshape(NW, NPAD, 1)
    g = _sage(v2e, agg, deg, Wself, Wneigh, bneigh.reshape(1, D))

    seq_flat = seq.reshape(SEQ_T).astype(jnp.int32)
    pos_flat = pos_idx.reshape(SEQ_T).astype(jnp.int32)
    uidx = (user[:, 0] + ITEM_NUM).astype(jnp.int32)
    seqg, posg, userg = _gathers(g, pos_table, seq_flat, pos_flat, uidx)

    maskf = mask.astype(jnp.float32).reshape(SEQ_T, 1)
    femb = _attn(
        seqg, posg, maskf, userg,
        w1[:D], w1[D:], glu1_W, glu1_b.reshape(1, D), glu2_W,
        w2.reshape(1, D), w3, glu3_W, glu3_b.reshape(1, D), glu4_W,
        w4.reshape(1, D), sc_W[:D].reshape(1, D), sc_W[D:].reshape(1, D),
        jnp.broadcast_to(sc_b.reshape(1, 1), (1, D)),
    )

    v2e_pad = jnp.pad(v2e, ((0, _NP - N), (0, 0)))
    scores = _scores(femb, v2e_pad)
    return scores[:, 1:N]


# speedup vs baseline: 3.2308x; 1.1911x over previous
import functools

import jax
import jax.numpy as jnp
from jax import lax
from jax.experimental import pallas as pl
from jax.experimental.pallas import tpu as pltpu
from jax.experimental.pallas import tpu_sc as plsc

N = 10000
E = 320000
D = 128
BS = 1024
L = 50
ITEM_NUM = 9000
POS = 200

NC = 2
NS = 16
NW = NC * NS
EPW = E // NW
CH = 80
NCH = EPW // CH
NPAIR = (NCH - 1) // 2
NPAD = 10240
RPT = NPAD // NS
RCH = RPT // CH

SEQ_T = BS * L
SPW = SEQ_T // NW
SCH = SPW // CH
GPAIR = SCH // 2
UPW = BS // NW


def _edge_agg_body(src_hbm, dst_hbm, v2e_hbm, z80_hbm,
                   agg_out, deg_out,
                   src0, src1, dst0, dst1, rows0, rows1, deg_v,
                   si0, si1, sg0, sg1, ss,
                   agg_sh):
    c = lax.axis_index("c")
    s = lax.axis_index("s")
    wid = s * NC + c
    base_r = s * RPT
    ebase = wid * EPW

    ones16 = jnp.ones((16,), jnp.float32)
    zero16 = jnp.zeros((16,), jnp.float32)

    pltpu.sync_copy(z80_hbm, rows0)

    def zbody(j, carry):
        pltpu.sync_copy(rows0, agg_sh.at[pl.ds(base_r + j * CH, CH)])
        return carry
    lax.fori_loop(0, RCH, zbody, 0)

    def zdeg(j, carry):
        deg_v[pl.ds(j * 16, 16)] = zero16
        return carry
    lax.fori_loop(0, NPAD // 16, zdeg, 0)
    plsc.subcore_barrier()

    def fetch_idx(i, sbuf, dbuf, sem):
        off = ebase + i * CH
        pltpu.async_copy(src_hbm.at[pl.ds(off, CH)], sbuf, sem)
        pltpu.async_copy(dst_hbm.at[pl.ds(off, CH)], dbuf, sem)

    def wait_idx(i, sbuf, dbuf, sem):
        off = ebase + i * CH
        pltpu.make_async_copy(src_hbm.at[pl.ds(off, CH)], sbuf, sem).wait()
        pltpu.make_async_copy(dst_hbm.at[pl.ds(off, CH)], dbuf, sem).wait()

    def wait_scatter(rbuf, dbuf):
        pltpu.make_async_copy(rbuf, agg_sh.at[dbuf], ss).wait()

    def deg_update(dbuf):
        for k in range(CH // 16):
            idx16 = dbuf[pl.ds(k * 16, 16)]
            plsc.addupdate_scatter(deg_v, [idx16], ones16)

    def half(i, sbuf, dbuf, rbuf, sem_i, sem_g,
             prev_rbuf, prev_dbuf, nxt_sbuf, nxt_dbuf, nxt_sem,
             wait_prev, prefetch):
        wait_idx(i, sbuf, dbuf, sem_i)
        gd = pltpu.async_copy(v2e_hbm.at[sbuf], rbuf, sem_g)
        if wait_prev:
            wait_scatter(prev_rbuf, prev_dbuf)
        gd.wait()
        if prefetch:
            fetch_idx(i + 1, nxt_sbuf, nxt_dbuf, nxt_sem)
        deg_update(dbuf)
        pltpu.async_copy(rbuf, agg_sh.at[dbuf], ss, add=True)

    fetch_idx(0, src0, dst0, si0)

    def pair(g, wait_first):
        i = 2 * g
        half(i, src0, dst0, rows0, si0, sg0,
             rows1, dst1, src1, dst1, si1, wait_first, True)
        half(i + 1, src1, dst1, rows1, si1, sg1,
             rows0, dst0, src0, dst0, si0, True, True)

    pair(0, False)

    def pbody(g, carry):
        pair(g, True)
        return carry
    lax.fori_loop(1, NPAIR, pbody, 0)

    half(NCH - 1, src0, dst0, rows0, si0, sg0,
         rows1, dst1, src1, dst1, si1, True, False)
    wait_scatter(rows0, dst0)
    plsc.subcore_barrier()

    out_r = c * NPAD + base_r

    def wbody(j, carry):
        pltpu.sync_copy(agg_sh.at[pl.ds(base_r + j * CH, CH)], rows0)
        pltpu.sync_copy(rows0, agg_out.at[pl.ds(out_r + j * CH, CH)])
        return carry
    lax.fori_loop(0, RCH, wbody, 0)
    pltpu.sync_copy(deg_v, deg_out.at[pl.ds(wid * NPAD, NPAD)])


def _edge_agg(src, dst, v2e, z80):
    return pl.kernel(
        _edge_agg_body,
        out_type=[
            jax.ShapeDtypeStruct((NC * NPAD, D), jnp.float32),
            jax.ShapeDtypeStruct((NW * NPAD,), jnp.float32),
        ],
        mesh=plsc.VectorSubcoreMesh(core_axis_name="c", subcore_axis_name="s",
                                    num_cores=NC, num_subcores=NS),
        scratch_types=[
            pltpu.VMEM((CH,), jnp.int32),
            pltpu.VMEM((CH,), jnp.int32),
            pltpu.VMEM((CH,), jnp.int32),
            pltpu.VMEM((CH,), jnp.int32),
            pltpu.VMEM((CH, D), jnp.float32),
            pltpu.VMEM((CH, D), jnp.float32),
            pltpu.VMEM((NPAD,), jnp.float32),
            pltpu.SemaphoreType.DMA,
            pltpu.SemaphoreType.DMA,
            pltpu.SemaphoreType.DMA,
            pltpu.SemaphoreType.DMA,
            pltpu.SemaphoreType.DMA,
            pltpu.VMEM_SHARED((NPAD, D), jnp.float32),
        ],
        compiler_params=pltpu.CompilerParams(needs_layout_passes=False),
    )(src, dst, v2e, z80)


def _gather_body(g_hbm, pos_tab_hbm, seq_hbm, pos_hbm, uidx_hbm,
                 seqg_out, posg_out, userg_out,
                 is0, is1, ip0, ip1, srows0, srows1, prows0, prows1,
                 uidx_v, urows_v,
                 csi0, csi1, cpi0, cpi1, csg0, csg1, cpg0, cpg1,
                 sws, swp, sem):
    c = lax.axis_index("c")
    s = lax.axis_index("s")
    wid = s * NC + c
    gbase = wid * SPW

    def fetch(i, ibuf, idx_hbm, sem_i):
        pltpu.async_copy(idx_hbm.at[pl.ds(gbase + i * CH, CH)], ibuf, sem_i)

    def wait_fetch(i, ibuf, idx_hbm, sem_i):
        pltpu.make_async_copy(idx_hbm.at[pl.ds(gbase + i * CH, CH)], ibuf,
                              sem_i).wait()

    def wait_wb(i, rbuf, out_hbm, sem_w):
        pltpu.make_async_copy(rbuf, out_hbm.at[pl.ds(gbase + i * CH, CH)],
                              sem_w).wait()

    def half(i, ibs, ibp, rbs, rbp, sis, sip, sgs, sgp,
             prev_rbs, prev_rbp, nxt_ibs, nxt_ibp, nxt_sis, nxt_sip,
             wait_prev, prefetch):
        wait_fetch(i, ibs, seq_hbm, sis)
        gs = pltpu.async_copy(g_hbm.at[ibs], rbs, sgs)
        wait_fetch(i, ibp, pos_hbm, sip)
        gp = pltpu.async_copy(pos_tab_hbm.at[ibp], rbp, sgp)
        if wait_prev:
            wait_wb(i - 1, prev_rbs, seqg_out, sws)
            wait_wb(i - 1, prev_rbp, posg_out, swp)
        gs.wait()
        pltpu.async_copy(rbs, seqg_out.at[pl.ds(gbase + i * CH, CH)], sws)
        gp.wait()
        pltpu.async_copy(rbp, posg_out.at[pl.ds(gbase + i * CH, CH)], swp)
        if prefetch:
            fetch(i + 1, nxt_ibs, seq_hbm, nxt_sis)
            fetch(i + 1, nxt_ibp, pos_hbm, nxt_sip)

    fetch(0, is0, seq_hbm, csi0)
    fetch(0, ip0, pos_hbm, cpi0)

    def pair(g, wait_first, prefetch_last):
        i = 2 * g
        half(i, is0, ip0, srows0, prows0, csi0, cpi0, csg0, cpg0,
             srows1, prows1, is1, ip1, csi1, cpi1, wait_first, True)
        half(i + 1, is1, ip1, srows1, prows1, csi1, cpi1, csg1, cpg1,
             srows0, prows0, is0, ip0, csi0, cpi0, True, prefetch_last)

    pair(0, False, True)

    def pbody(g, carry):
        pair(g, True, True)
        return carry
    lax.fori_loop(1, GPAIR - 1, pbody, 0)
    pair(GPAIR - 1, True, False)
    wait_wb(SCH - 1, srows1, seqg_out, sws)
    wait_wb(SCH - 1, prows1, posg_out, swp)

    uoff = wid * UPW
    pltpu.sync_copy(uidx_hbm.at[pl.ds(uoff, UPW)], uidx_v)
    pltpu.async_copy(g_hbm.at[uidx_v], urows_v, sem).wait()
    pltpu.sync_copy(urows_v, userg_out.at[pl.ds(uoff, UPW)])


def _gathers(g, pos_table, seq_flat, pos_flat, uidx):
    return pl.kernel(
        _gather_body,
        out_type=[
            jax.ShapeDtypeStruct((SEQ_T, D), jnp.float32),
            jax.ShapeDtypeStruct((SEQ_T, D), jnp.float32),
            jax.ShapeDtypeStruct((BS, D), jnp.float32),
        ],
        mesh=plsc.VectorSubcoreMesh(core_axis_name="c", subcore_axis_name="s",
                                    num_cores=NC, num_subcores=NS),
        scratch_types=[
            pltpu.VMEM((CH,), jnp.int32),
            pltpu.VMEM((CH,), jnp.int32),
            pltpu.VMEM((CH,), jnp.int32),
            pltpu.VMEM((CH,), jnp.int32),
            pltpu.VMEM((CH, D), jnp.float32),
            pltpu.VMEM((CH, D), jnp.float32),
            pltpu.VMEM((CH, D), jnp.float32),
            pltpu.VMEM((CH, D), jnp.float32),
            pltpu.VMEM((UPW,), jnp.int32),
            pltpu.VMEM((UPW, D), jnp.float32),
            pltpu.SemaphoreType.DMA,
            pltpu.SemaphoreType.DMA,
            pltpu.SemaphoreType.DMA,
            pltpu.SemaphoreType.DMA,
            pltpu.SemaphoreType.DMA,
            pltpu.SemaphoreType.DMA,
            pltpu.SemaphoreType.DMA,
            pltpu.SemaphoreType.DMA,
            pltpu.SemaphoreType.DMA,
            pltpu.SemaphoreType.DMA,
            pltpu.SemaphoreType.DMA,
        ],
    )(g, pos_table, seq_flat, pos_flat, uidx)


_BN = 400


def _sage_body(v2e_ref, agg_ref, deg_ref, ws_ref, wn_ref, b_ref, g_ref):
    x = v2e_ref[...]
    a = agg_ref[0] + agg_ref[1]
    d = jnp.sum(deg_ref[...], axis=0)
    neigh = a / jnp.maximum(d, 1.0)
    h = jnp.dot(x, ws_ref[...], preferred_element_type=jnp.float32)
    h = h + jnp.dot(neigh, wn_ref[...], preferred_element_type=jnp.float32)
    h = jax.nn.relu(h + b_ref[...])
    g_ref[...] = 0.5 * (h + x)


def _sage(v2e, agg, deg, Wself, Wneigh, bneigh):
    return pl.pallas_call(
        _sage_body,
        grid=(N // _BN,),
        in_specs=[
            pl.BlockSpec((_BN, D), lambda i: (i, 0)),
            pl.BlockSpec((NC, _BN, D), lambda i: (0, i, 0)),
            pl.BlockSpec((NW, _BN, 1), lambda i: (0, i, 0)),
            pl.BlockSpec((D, D), lambda i: (0, 0)),
            pl.BlockSpec((D, D), lambda i: (0, 0)),
            pl.BlockSpec((1, D), lambda i: (0, 0)),
        ],
        out_specs=pl.BlockSpec((_BN, D), lambda i: (i, 0)),
        out_shape=jax.ShapeDtypeStruct((N, D), jnp.float32),
    )(v2e, agg, deg, Wself, Wneigh, bneigh)


_BB = 8
_BM = _BB * L


def _attn_body(node_ref, pos_ref, m_ref, u_ref,
               w1a_ref, w1b_ref, g1w_ref, g1b_ref, g2w_ref, w2_ref,
               w3_ref, g3w_ref, g3b_ref, g4w_ref, w4_ref,
               scwa_ref, scwb_ref, scb_ref, out_ref):
    node = node_ref[...]
    posv = pos_ref[...]
    m = m_ref[...]
    u = u_ref[...]

    row = lax.broadcasted_iota(jnp.int32, (_BM, _BB), 0) // L
    col = lax.broadcasted_iota(jnp.int32, (_BM, _BB), 1)
    selT = (row == col).astype(jnp.float32)

    def seg_sum(x):
        return lax.dot_general(selT, x, (((0,), (0,)), ((), ())),
                               preferred_element_type=jnp.float32)

    def expand(x):
        return jnp.dot(selT, x, preferred_element_type=jnp.float32)

    mnode = node * m
    tmp = seg_sum(mnode) / seg_sum(m)
    hsb = expand(jnp.dot(tmp, g2w_ref[...],
                         preferred_element_type=jnp.float32))

    nh = jnp.tanh(jnp.dot(posv, w1a_ref[...], preferred_element_type=jnp.float32)
                  + jnp.dot(node, w1b_ref[...], preferred_element_type=jnp.float32))
    nh = jax.nn.sigmoid(jnp.dot(nh, g1w_ref[...], preferred_element_type=jnp.float32)
                        + g1b_ref[...] + hsb)
    beta = jnp.sum(nh * w2_ref[...], axis=-1, keepdims=True) * m
    sess = seg_sum(beta * node)

    ub = expand(jnp.dot(u, g4w_ref[...], preferred_element_type=jnp.float32))
    nh2 = jnp.tanh(jnp.dot(node, w3_ref[...], preferred_element_type=jnp.float32))
    nh2 = jax.nn.sigmoid(jnp.dot(nh2, g3w_ref[...], preferred_element_type=jnp.float32)
                         + g3b_ref[...] + ub)
    beta2 = jnp.sum(nh2 * w4_ref[...], axis=-1, keepdims=True) * m
    sessu = seg_sum(beta2 * node)

    a1 = jnp.sum(sess * scwa_ref[...], axis=-1, keepdims=True)
    a2 = jnp.sum(sessu * scwb_ref[...], axis=-1, keepdims=True)
    alpha = jax.nn.sigmoid(a1 + a2 + scb_ref[0:1, 0:1])
    out_ref[...] = u + alpha * sess + (1.0 - alpha) * sessu


def _attn(seqg, posg, maskf, userg, w1a, w1b, g1w, g1b, g2w, w2r,
          w3, g3w, g3b, g4w, w4r, scwa, scwb, scb):
    full = lambda shape: pl.BlockSpec(shape, lambda i: tuple(0 for _ in shape))
    return pl.pallas_call(
        _attn_body,
        grid=(BS // _BB,),
        in_specs=[
            pl.BlockSpec((_BM, D), lambda i: (i, 0)),
            pl.BlockSpec((_BM, D), lambda i: (i, 0)),
            pl.BlockSpec((_BM, 1), lambda i: (i, 0)),
            pl.BlockSpec((_BB, D), lambda i: (i, 0)),
            full((D, D)), full((D, D)), full((D, D)), full((1, D)),
            full((D, D)), full((1, D)),
            full((D, D)), full((D, D)), full((1, D)), full((D, D)),
            full((1, D)), full((1, D)), full((1, D)), full((1, D)),
        ],
        out_specs=pl.BlockSpec((_BB, D), lambda i: (i, 0)),
        out_shape=jax.ShapeDtypeStruct((BS, D), jnp.float32),
    )(seqg, posg, maskf, userg, w1a, w1b, g1w, g1b, g2w, w2r,
      w3, g3w, g3b, g4w, w4r, scwa, scwb, scb)


_NV = N - 1
_BV = 1280


def _scores_body(f_ref, v_ref, o_ref):
    o_ref[...] = lax.dot_general(f_ref[...], v_ref[...],
                                 (((1,), (1,)), ((), ())),
                                 preferred_element_type=jnp.float32)


def _scores(femb, v2e_sl):
    return pl.pallas_call(
        _scores_body,
        grid=(pl.cdiv(_NV, _BV),),
        in_specs=[
            pl.BlockSpec((BS, D), lambda i: (0, 0)),
            pl.BlockSpec((_BV, D), lambda i: (i, 0)),
        ],
        out_specs=pl.BlockSpec((BS, _BV), lambda i: (0, i)),
        out_shape=jax.ShapeDtypeStruct((BS, _NV), jnp.float32),
    )(femb, v2e_sl)


def kernel(user, seq, mask, seq_len, pos_idx, edge_index, v2e, pos_table,
           Wself, Wneigh, bneigh, w1, w2, glu1_W, glu1_b, glu2_W, w3, w4,
           glu3_W, glu3_b, glu4_W, sc_W, sc_b):
    src = edge_index[0].astype(jnp.int32)
    dst = edge_index[1].astype(jnp.int32)
    z80 = jnp.zeros((CH, D), jnp.float32)

    agg, deg = _edge_agg(src, dst, v2e, z80)
    agg = agg.reshape(NC, NPAD, D)
    deg = deg.reshape(NW, NPAD, 1)
    g = _sage(v2e, agg, deg, Wself, Wneigh, bneigh.reshape(1, D))

    seq_flat = seq.reshape(SEQ_T).astype(jnp.int32)
    pos_flat = pos_idx.reshape(SEQ_T).astype(jnp.int32)
    uidx = (user[:, 0] + ITEM_NUM).astype(jnp.int32)
    seqg, posg, userg = _gathers(g, pos_table, seq_flat, pos_flat, uidx)

    maskf = mask.astype(jnp.float32).reshape(SEQ_T, 1)
    femb = _attn(
        seqg, posg, maskf, userg,
        w1[:D], w1[D:], glu1_W, glu1_b.reshape(1, D), glu2_W,
        w2.reshape(1, D), w3, glu3_W, glu3_b.reshape(1, D), glu4_W,
        w4.reshape(1, D), sc_W[:D].reshape(1, D), sc_W[D:].reshape(1, D),
        jnp.broadcast_to(sc_b.reshape(1, 1), (1, D)),
    )

    return _scores(femb, v2e[1:])


# speedup vs baseline: 3.7350x; 1.1560x over previous
import functools

import jax
import jax.numpy as jnp
from jax import lax
from jax.experimental import pallas as pl
from jax.experimental.pallas import tpu as pltpu
from jax.experimental.pallas import tpu_sc as plsc

N = 10000
E = 320000
D = 128
BS = 1024
L = 50
ITEM_NUM = 9000
POS = 200

NC = 2
NS = 16
NW = NC * NS
EPW = E // NW
CH = 80
NCH = EPW // CH
NPAIR = (NCH - 1) // 2
NPAD = 10240
RPT = NPAD // NS
RCH = RPT // CH

SEQ_T = BS * L
SPW = SEQ_T // NW
SCH = SPW // CH
GPAIR = SCH // 2
UPW = BS // NW


def _edge_agg_body(src_hbm, dst_hbm, v2e_hbm, z80_hbm,
                   agg_out, deg_out,
                   src0, src1, src2, dst0, dst1, dst2, sd0, sd1, sd2,
                   rows0, rows1, rows2, deg_v,
                   si0, si1, si2, sg0, sg1, sg2, ss,
                   agg_sh):
    c = lax.axis_index("c")
    s = lax.axis_index("s")
    wid = s * NC + c
    base_r = s * RPT
    ebase = wid * EPW

    srcb = (src0, src1, src2)
    dstb = (dst0, dst1, dst2)
    rowb = (rows0, rows1, rows2)
    sib = (si0, si1, si2)
    sgb = (sg0, sg1, sg2)
    sdb = (sd0, sd1, sd2)

    ones16 = jnp.ones((16,), jnp.float32)
    zero16 = jnp.zeros((16,), jnp.float32)

    pltpu.sync_copy(z80_hbm, rows0)

    def zbody(j, carry):
        pltpu.sync_copy(rows0, agg_sh.at[pl.ds(base_r + j * CH, CH)])
        return carry
    lax.fori_loop(0, RCH, zbody, 0)

    def zdeg(j, carry):
        deg_v[pl.ds(j * 16, 16)] = zero16
        return carry
    lax.fori_loop(0, NPAD // 16, zdeg, 0)
    plsc.subcore_barrier()

    def fetch_idx(i, b):
        off = ebase + i * CH
        pltpu.async_copy(src_hbm.at[pl.ds(off, CH)], srcb[b], sib[b])
        pltpu.async_copy(dst_hbm.at[pl.ds(off, CH)], dstb[b], sib[b])

    def wait_idx(i, b):
        off = ebase + i * CH
        pltpu.make_async_copy(src_hbm.at[pl.ds(off, CH)], srcb[b],
                              sib[b]).wait()
        pltpu.make_async_copy(dst_hbm.at[pl.ds(off, CH)], dstb[b],
                              sib[b]).wait()

    def wait_scatter(b3):
        pltpu.make_async_copy(rowb[b3], agg_sh.at[sdb[b3]], ss).wait()

    def body(i, b3, wait_sc, do_fetch, do_gather):
        if wait_sc:
            wait_scatter((b3 - 2) % 3)
        if do_fetch:
            fetch_idx(i + 2, (b3 + 2) % 3)
        if do_gather:
            nb = (b3 + 1) % 3
            wait_idx(i + 1, nb)
            pltpu.async_copy(v2e_hbm.at[srcb[nb]], rowb[nb], sgb[nb])
        pltpu.make_async_copy(v2e_hbm.at[srcb[b3]], rowb[b3],
                              sgb[b3]).wait()
        for k in range(CH // 16):
            idx16 = dstb[b3][pl.ds(k * 16, 16)]
            sdb[b3][pl.ds(k * 16, 16)] = idx16
            plsc.addupdate_scatter(deg_v, [idx16], ones16)
        pltpu.async_copy(rowb[b3], agg_sh.at[sdb[b3]], ss, add=True)

    fetch_idx(0, 0)
    fetch_idx(1, 1)
    wait_idx(0, 0)
    pltpu.async_copy(v2e_hbm.at[src0], rows0, sg0)
    body(0, 0, False, True, True)
    body(1, 1, False, True, True)

    def pbody(g, carry):
        i = 3 * g + 2
        body(i, 2, True, True, True)
        body(i + 1, 0, True, True, True)
        body(i + 2, 1, True, True, True)
        return carry
    lax.fori_loop(0, (NCH - 5) // 3, pbody, 0)

    body(NCH - 3, 2, True, True, True)
    body(NCH - 2, 0, True, False, True)
    body(NCH - 1, 1, True, False, False)
    wait_scatter(0)
    wait_scatter(1)
    plsc.subcore_barrier()

    out_r = c * NPAD + base_r

    def wbody(j, carry):
        pltpu.sync_copy(agg_sh.at[pl.ds(base_r + j * CH, CH)], rows0)
        pltpu.sync_copy(rows0, agg_out.at[pl.ds(out_r + j * CH, CH)])
        return carry
    lax.fori_loop(0, RCH, wbody, 0)
    pltpu.sync_copy(deg_v, deg_out.at[pl.ds(wid * NPAD, NPAD)])


def _edge_agg(src, dst, v2e, z80):
    return pl.kernel(
        _edge_agg_body,
        out_type=[
            jax.ShapeDtypeStruct((NC * NPAD, D), jnp.float32),
            jax.ShapeDtypeStruct((NW * NPAD,), jnp.float32),
        ],
        mesh=plsc.VectorSubcoreMesh(core_axis_name="c", subcore_axis_name="s",
                                    num_cores=NC, num_subcores=NS),
        scratch_types=[
            pltpu.VMEM((CH,), jnp.int32),
            pltpu.VMEM((CH,), jnp.int32),
            pltpu.VMEM((CH,), jnp.int32),
            pltpu.VMEM((CH,), jnp.int32),
            pltpu.VMEM((CH,), jnp.int32),
            pltpu.VMEM((CH,), jnp.int32),
            pltpu.VMEM((CH,), jnp.int32),
            pltpu.VMEM((CH,), jnp.int32),
            pltpu.VMEM((CH,), jnp.int32),
            pltpu.VMEM((CH, D), jnp.float32),
            pltpu.VMEM((CH, D), jnp.float32),
            pltpu.VMEM((CH, D), jnp.float32),
            pltpu.VMEM((NPAD,), jnp.float32),
            pltpu.SemaphoreType.DMA,
            pltpu.SemaphoreType.DMA,
            pltpu.SemaphoreType.DMA,
            pltpu.SemaphoreType.DMA,
            pltpu.SemaphoreType.DMA,
            pltpu.SemaphoreType.DMA,
            pltpu.SemaphoreType.DMA,
            pltpu.VMEM_SHARED((NPAD, D), jnp.float32),
        ],
        compiler_params=pltpu.CompilerParams(needs_layout_passes=False),
    )(src, dst, v2e, z80)


def _gather_body(g_hbm, pos_tab_hbm, seq_hbm, pos_hbm, uidx_hbm,
                 seqg_out, posg_out, userg_out,
                 is0, is1, ip0, ip1, srows0, srows1, prows0, prows1,
                 uidx_v, urows_v,
                 csi0, csi1, cpi0, cpi1, csg0, csg1, cpg0, cpg1,
                 sws, swp, sem):
    c = lax.axis_index("c")
    s = lax.axis_index("s")
    wid = s * NC + c
    gbase = wid * SPW

    def fetch(i, ibuf, idx_hbm, sem_i):
        pltpu.async_copy(idx_hbm.at[pl.ds(gbase + i * CH, CH)], ibuf, sem_i)

    def wait_fetch(i, ibuf, idx_hbm, sem_i):
        pltpu.make_async_copy(idx_hbm.at[pl.ds(gbase + i * CH, CH)], ibuf,
                              sem_i).wait()

    def wait_wb(i, rbuf, out_hbm, sem_w):
        pltpu.make_async_copy(rbuf, out_hbm.at[pl.ds(gbase + i * CH, CH)],
                              sem_w).wait()

    def half(i, ibs, ibp, rbs, rbp, sis, sip, sgs, sgp,
             prev_rbs, prev_rbp, nxt_ibs, nxt_ibp, nxt_sis, nxt_sip,
             wait_prev, prefetch):
        wait_fetch(i, ibs, seq_hbm, sis)
        gs = pltpu.async_copy(g_hbm.at[ibs], rbs, sgs)
        wait_fetch(i, ibp, pos_hbm, sip)
        gp = pltpu.async_copy(pos_tab_hbm.at[ibp], rbp, sgp)
        if wait_prev:
            wait_wb(i - 1, prev_rbs, seqg_out, sws)
            wait_wb(i - 1, prev_rbp, posg_out, swp)
        gs.wait()
        pltpu.async_copy(rbs, seqg_out.at[pl.ds(gbase + i * CH, CH)], sws)
        gp.wait()
        pltpu.async_copy(rbp, posg_out.at[pl.ds(gbase + i * CH, CH)], swp)
        if prefetch:
            fetch(i + 1, nxt_ibs, seq_hbm, nxt_sis)
            fetch(i + 1, nxt_ibp, pos_hbm, nxt_sip)

    fetch(0, is0, seq_hbm, csi0)
    fetch(0, ip0, pos_hbm, cpi0)

    def pair(g, wait_first, prefetch_last):
        i = 2 * g
        half(i, is0, ip0, srows0, prows0, csi0, cpi0, csg0, cpg0,
             srows1, prows1, is1, ip1, csi1, cpi1, wait_first, True)
        half(i + 1, is1, ip1, srows1, prows1, csi1, cpi1, csg1, cpg1,
             srows0, prows0, is0, ip0, csi0, cpi0, True, prefetch_last)

    pair(0, False, True)

    def pbody(g, carry):
        pair(g, True, True)
        return carry
    lax.fori_loop(1, GPAIR - 1, pbody, 0)
    pair(GPAIR - 1, True, False)
    wait_wb(SCH - 1, srows1, seqg_out, sws)
    wait_wb(SCH - 1, prows1, posg_out, swp)

    uoff = wid * UPW
    pltpu.sync_copy(uidx_hbm.at[pl.ds(uoff, UPW)], uidx_v)
    pltpu.async_copy(g_hbm.at[uidx_v], urows_v, sem).wait()
    pltpu.sync_copy(urows_v, userg_out.at[pl.ds(uoff, UPW)])


def _gathers(g, pos_table, seq_flat, pos_flat, uidx):
    return pl.kernel(
        _gather_body,
        out_type=[
            jax.ShapeDtypeStruct((SEQ_T, D), jnp.float32),
            jax.ShapeDtypeStruct((SEQ_T, D), jnp.float32),
            jax.ShapeDtypeStruct((BS, D), jnp.float32),
        ],
        mesh=plsc.VectorSubcoreMesh(core_axis_name="c", subcore_axis_name="s",
                                    num_cores=NC, num_subcores=NS),
        scratch_types=[
            pltpu.VMEM((CH,), jnp.int32),
            pltpu.VMEM((CH,), jnp.int32),
            pltpu.VMEM((CH,), jnp.int32),
            pltpu.VMEM((CH,), jnp.int32),
            pltpu.VMEM((CH, D), jnp.float32),
            pltpu.VMEM((CH, D), jnp.float32),
            pltpu.VMEM((CH, D), jnp.float32),
            pltpu.VMEM((CH, D), jnp.float32),
            pltpu.VMEM((UPW,), jnp.int32),
            pltpu.VMEM((UPW, D), jnp.float32),
            pltpu.SemaphoreType.DMA,
            pltpu.SemaphoreType.DMA,
            pltpu.SemaphoreType.DMA,
            pltpu.SemaphoreType.DMA,
            pltpu.SemaphoreType.DMA,
            pltpu.SemaphoreType.DMA,
            pltpu.SemaphoreType.DMA,
            pltpu.SemaphoreType.DMA,
            pltpu.SemaphoreType.DMA,
            pltpu.SemaphoreType.DMA,
            pltpu.SemaphoreType.DMA,
        ],
    )(g, pos_table, seq_flat, pos_flat, uidx)


_BN = 400


def _sage_body(v2e_ref, agg_ref, deg_ref, ws_ref, wn_ref, b_ref, g_ref):
    x = v2e_ref[...]
    a = agg_ref[0] + agg_ref[1]
    d = jnp.sum(deg_ref[...], axis=0)
    neigh = a / jnp.maximum(d, 1.0)
    h = jnp.dot(x, ws_ref[...], preferred_element_type=jnp.float32)
    h = h + jnp.dot(neigh, wn_ref[...], preferred_element_type=jnp.float32)
    h = jax.nn.relu(h + b_ref[...])
    g_ref[...] = 0.5 * (h + x)


def _sage(v2e, agg, deg, Wself, Wneigh, bneigh):
    return pl.pallas_call(
        _sage_body,
        grid=(N // _BN,),
        in_specs=[
            pl.BlockSpec((_BN, D), lambda i: (i, 0)),
            pl.BlockSpec((NC, _BN, D), lambda i: (0, i, 0)),
            pl.BlockSpec((NW, _BN, 1), lambda i: (0, i, 0)),
            pl.BlockSpec((D, D), lambda i: (0, 0)),
            pl.BlockSpec((D, D), lambda i: (0, 0)),
            pl.BlockSpec((1, D), lambda i: (0, 0)),
        ],
        out_specs=pl.BlockSpec((_BN, D), lambda i: (i, 0)),
        out_shape=jax.ShapeDtypeStruct((N, D), jnp.float32),
    )(v2e, agg, deg, Wself, Wneigh, bneigh)


_BB = 8
_BM = _BB * L


def _attn_body(node_ref, pos_ref, m_ref, u_ref,
               w1a_ref, w1b_ref, g1w_ref, g1b_ref, g2w_ref, w2_ref,
               w3_ref, g3w_ref, g3b_ref, g4w_ref, w4_ref,
               scwa_ref, scwb_ref, scb_ref, out_ref):
    node = node_ref[...]
    posv = pos_ref[...]
    m = m_ref[...]
    u = u_ref[...]

    row = lax.broadcasted_iota(jnp.int32, (_BM, _BB), 0) // L
    col = lax.broadcasted_iota(jnp.int32, (_BM, _BB), 1)
    selT = (row == col).astype(jnp.float32)

    def seg_sum(x):
        return lax.dot_general(selT, x, (((0,), (0,)), ((), ())),
                               preferred_element_type=jnp.float32)

    def expand(x):
        return jnp.dot(selT, x, preferred_element_type=jnp.float32)

    mnode = node * m
    tmp = seg_sum(mnode) / seg_sum(m)
    hsb = expand(jnp.dot(tmp, g2w_ref[...],
                         preferred_element_type=jnp.float32))

    nh = jnp.tanh(jnp.dot(posv, w1a_ref[...], preferred_element_type=jnp.float32)
                  + jnp.dot(node, w1b_ref[...], preferred_element_type=jnp.float32))
    nh = jax.nn.sigmoid(jnp.dot(nh, g1w_ref[...], preferred_element_type=jnp.float32)
                        + g1b_ref[...] + hsb)
    beta = jnp.sum(nh * w2_ref[...], axis=-1, keepdims=True) * m
    sess = seg_sum(beta * node)

    ub = expand(jnp.dot(u, g4w_ref[...], preferred_element_type=jnp.float32))
    nh2 = jnp.tanh(jnp.dot(node, w3_ref[...], preferred_element_type=jnp.float32))
    nh2 = jax.nn.sigmoid(jnp.dot(nh2, g3w_ref[...], preferred_element_type=jnp.float32)
                         + g3b_ref[...] + ub)
    beta2 = jnp.sum(nh2 * w4_ref[...], axis=-1, keepdims=True) * m
    sessu = seg_sum(beta2 * node)

    a1 = jnp.sum(sess * scwa_ref[...], axis=-1, keepdims=True)
    a2 = jnp.sum(sessu * scwb_ref[...], axis=-1, keepdims=True)
    alpha = jax.nn.sigmoid(a1 + a2 + scb_ref[0:1, 0:1])
    out_ref[...] = u + alpha * sess + (1.0 - alpha) * sessu


def _attn(seqg, posg, maskf, userg, w1a, w1b, g1w, g1b, g2w, w2r,
          w3, g3w, g3b, g4w, w4r, scwa, scwb, scb):
    full = lambda shape: pl.BlockSpec(shape, lambda i: tuple(0 for _ in shape))
    return pl.pallas_call(
        _attn_body,
        grid=(BS // _BB,),
        in_specs=[
            pl.BlockSpec((_BM, D), lambda i: (i, 0)),
            pl.BlockSpec((_BM, D), lambda i: (i, 0)),
            pl.BlockSpec((_BM, 1), lambda i: (i, 0)),
            pl.BlockSpec((_BB, D), lambda i: (i, 0)),
            full((D, D)), full((D, D)), full((D, D)), full((1, D)),
            full((D, D)), full((1, D)),
            full((D, D)), full((D, D)), full((1, D)), full((D, D)),
            full((1, D)), full((1, D)), full((1, D)), full((1, D)),
        ],
        out_specs=pl.BlockSpec((_BB, D), lambda i: (i, 0)),
        out_shape=jax.ShapeDtypeStruct((BS, D), jnp.float32),
    )(seqg, posg, maskf, userg, w1a, w1b, g1w, g1b, g2w, w2r,
      w3, g3w, g3b, g4w, w4r, scwa, scwb, scb)


_NV = N - 1
_BV = 1280


def _scores_body(f_ref, v_ref, o_ref):
    o_ref[...] = lax.dot_general(f_ref[...], v_ref[...],
                                 (((1,), (1,)), ((), ())),
                                 preferred_element_type=jnp.float32)


def _scores(femb, v2e_sl):
    return pl.pallas_call(
        _scores_body,
        grid=(pl.cdiv(_NV, _BV),),
        in_specs=[
            pl.BlockSpec((BS, D), lambda i: (0, 0)),
            pl.BlockSpec((_BV, D), lambda i: (i, 0)),
        ],
        out_specs=pl.BlockSpec((BS, _BV), lambda i: (0, i)),
        out_shape=jax.ShapeDtypeStruct((BS, _NV), jnp.float32),
    )(femb, v2e_sl)


def kernel(user, seq, mask, seq_len, pos_idx, edge_index, v2e, pos_table,
           Wself, Wneigh, bneigh, w1, w2, glu1_W, glu1_b, glu2_W, w3, w4,
           glu3_W, glu3_b, glu4_W, sc_W, sc_b):
    src = edge_index[0].astype(jnp.int32)
    dst = edge_index[1].astype(jnp.int32)
    z80 = jnp.zeros((CH, D), jnp.float32)

    agg, deg = _edge_agg(src, dst, v2e, z80)
    agg = agg.reshape(NC, NPAD, D)
    deg = deg.reshape(NW, NPAD, 1)
    g = _sage(v2e, agg, deg, Wself, Wneigh, bneigh.reshape(1, D))

    seq_flat = seq.reshape(SEQ_T).astype(jnp.int32)
    pos_flat = pos_idx.reshape(SEQ_T).astype(jnp.int32)
    uidx = (user[:, 0] + ITEM_NUM).astype(jnp.int32)
    seqg, posg, userg = _gathers(g, pos_table, seq_flat, pos_flat, uidx)

    maskf = mask.astype(jnp.float32).reshape(SEQ_T, 1)
    femb = _attn(
        seqg, posg, maskf, userg,
        w1[:D], w1[D:], glu1_W, glu1_b.reshape(1, D), glu2_W,
        w2.reshape(1, D), w3, glu3_W, glu3_b.reshape(1, D), glu4_W,
        w4.reshape(1, D), sc_W[:D].reshape(1, D), sc_W[D:].reshape(1, D),
        jnp.broadcast_to(sc_b.reshape(1, 1), (1, D)),
    )

    return _scores(femb, v2e[1:])
